# baseline (speedup 1.0000x reference)
import functools

import jax
import jax.numpy as jnp
from jax.experimental import pallas as pl
from jax.experimental.pallas import tpu as pltpu

TOPK = 2
BT = 256
BM = 128


def _gate_kernel(h_ref, wg_ref, e1_ref, e2_ref, w1_ref, w2_ref,
                 p1_ref, p2_ref, loadw_ref, cnt_ref, aux_ref):
    g = pl.program_id(0)
    ng = pl.num_programs(0)
    E = wg_ref.shape[1]
    logits = jnp.dot(h_ref[...], wg_ref[...],
                     preferred_element_type=jnp.float32)
    eidx = jax.lax.broadcasted_iota(jnp.int32, logits.shape, 1)
    m1 = jnp.max(logits, axis=1, keepdims=True)
    a1 = jnp.min(jnp.where(logits == m1, eidx, E), axis=1, keepdims=True)
    masked = jnp.where(eidx == a1, -jnp.inf, logits)
    m2 = jnp.max(masked, axis=1, keepdims=True)
    a2 = jnp.min(jnp.where(masked == m2, eidx, E), axis=1, keepdims=True)
    p1 = 1.0 / (1.0 + jnp.exp(m2 - m1))
    p2 = 1.0 - p1
    e1_ref[...] = a1
    e2_ref[...] = a2
    w1_ref[...] = p1
    w2_ref[...] = p2
    one1 = (eidx == a1).astype(jnp.float32)
    one2 = (eidx == a2).astype(jnp.float32)
    both = one1 + one2
    ridx = jax.lax.broadcasted_iota(jnp.int32, (h_ref.shape[0],) * 2, 0)
    cidx = jax.lax.broadcasted_iota(jnp.int32, (h_ref.shape[0],) * 2, 1)
    tri = (cidx < ridx).astype(jnp.float32)
    csum = jnp.dot(tri, both, preferred_element_type=jnp.float32)
    carry = jnp.where(g == 0, 0.0, cnt_ref[...])
    total = carry + csum
    p1_ref[...] = jnp.sum(one1 * total, axis=1, keepdims=True).astype(
        jnp.int32)
    p2_ref[...] = jnp.sum(one2 * total, axis=1, keepdims=True).astype(
        jnp.int32)
    cnt_ref[...] = carry + jnp.sum(both, axis=0, keepdims=True)
    loadc = jnp.sum(one1 * p1 + one2 * p2, axis=0, keepdims=True)

    @pl.when(g == 0)
    def _():
        loadw_ref[...] = loadc

    @pl.when(g != 0)
    def _():
        loadw_ref[...] += loadc

    @pl.when(g == ng - 1)
    def _():
        load = loadw_ref[...]
        ln = load / jnp.sum(load)
        aux_ref[...] = jnp.sum(ln * jnp.log(ln + 1e-9)).reshape(1, 1)


def _ffn_kernel(se_ref, sb_ref, ss_ref, sen_ref, x_ref, w1_ref, w2_ref,
                o_ref):
    g = pl.program_id(0)
    b = sb_ref[g]
    start = ss_ref[g]
    end = sen_ref[g]
    r = b * BM + jax.lax.broadcasted_iota(jnp.int32, (BM, 1), 0)
    valid = (r >= start) & (r < end)
    xb = x_ref[...]
    h1 = jnp.dot(xb, w1_ref[0], preferred_element_type=jnp.float32)
    a = h1 * jax.nn.sigmoid(h1)
    z = jnp.dot(a, w2_ref[0], preferred_element_type=jnp.float32)
    z = jnp.where(valid, z, 0.0)
    first = jnp.logical_or(g == 0, sb_ref[jnp.maximum(g - 1, 0)] != b)

    @pl.when(first)
    def _():
        o_ref[...] = z

    @pl.when(jnp.logical_not(first))
    def _():
        o_ref[...] += z


def kernel(x, Wg, W1, W2):
    b, t, d = x.shape
    h = x.reshape(-1, d)
    N = h.shape[0]
    E = Wg.shape[1]
    DFF = W1.shape[2]
    P = N * TOPK
    NB = P // BM
    G = NB + E - 1

    ngate = N // BT
    e1, e2, w1, w2, pp1, pp2, loadw, cnt, aux = pl.pallas_call(
        _gate_kernel,
        grid=(ngate,),
        in_specs=[
            pl.BlockSpec((BT, d), lambda g: (g, 0)),
            pl.BlockSpec((d, E), lambda g: (0, 0)),
        ],
        out_specs=[
            pl.BlockSpec((BT, 1), lambda g: (g, 0)),
            pl.BlockSpec((BT, 1), lambda g: (g, 0)),
            pl.BlockSpec((BT, 1), lambda g: (g, 0)),
            pl.BlockSpec((BT, 1), lambda g: (g, 0)),
            pl.BlockSpec((BT, 1), lambda g: (g, 0)),
            pl.BlockSpec((BT, 1), lambda g: (g, 0)),
            pl.BlockSpec((1, E), lambda g: (0, 0)),
            pl.BlockSpec((1, E), lambda g: (0, 0)),
            pl.BlockSpec((1, 1), lambda g: (0, 0)),
        ],
        out_shape=[
            jax.ShapeDtypeStruct((N, 1), jnp.int32),
            jax.ShapeDtypeStruct((N, 1), jnp.int32),
            jax.ShapeDtypeStruct((N, 1), jnp.float32),
            jax.ShapeDtypeStruct((N, 1), jnp.float32),
            jax.ShapeDtypeStruct((N, 1), jnp.int32),
            jax.ShapeDtypeStruct((N, 1), jnp.int32),
            jax.ShapeDtypeStruct((1, E), jnp.float32),
            jax.ShapeDtypeStruct((1, E), jnp.float32),
            jax.ShapeDtypeStruct((1, 1), jnp.float32),
        ],
    )(h, Wg)

    counts = cnt[0].astype(jnp.int32)
    off = jnp.concatenate([jnp.zeros((1,), jnp.int32),
                           jnp.cumsum(counts)]).astype(jnp.int32)

    ep = jnp.concatenate([e1, e2], axis=1)
    wp = jnp.concatenate([w1, w2], axis=1).reshape(-1)
    posl = jnp.concatenate([pp1, pp2], axis=1)
    inv = (off[ep] + posl).reshape(-1)
    tok_sorted = jnp.zeros((P,), jnp.int32).at[inv].set(
        jnp.arange(P, dtype=jnp.int32) // TOPK)
    x_sorted = jnp.take(h, tok_sorted, axis=0)

    first_blk = off[:-1] // BM
    last_blk = jnp.maximum(off[1:] - 1, 0) // BM
    nb = jnp.where(counts > 0, last_blk - first_blk + 1, 0)
    cum = jnp.cumsum(nb)
    total = cum[-1]
    gidx = jnp.arange(G, dtype=jnp.int32)
    eg = jnp.searchsorted(cum, gidx, side='right').astype(jnp.int32)
    eg = jnp.minimum(eg, E - 1)
    cum0 = jnp.concatenate([jnp.zeros((1,), jnp.int32),
                            cum.astype(jnp.int32)])
    j = gidx - cum0[eg]
    blk = first_blk[eg] + j
    live = gidx < total
    last_real = jnp.maximum(total - 1, 0)
    se = jnp.where(live, eg, eg[last_real]).astype(jnp.int32)
    sb = jnp.where(live, blk, blk[last_real]).astype(jnp.int32)
    ss = jnp.where(live, jnp.maximum(off[se], sb * BM), 0).astype(jnp.int32)
    sen = jnp.where(live, jnp.minimum(off[se + 1], (sb + 1) * BM),
                    0).astype(jnp.int32)

    out_sorted = pl.pallas_call(
        _ffn_kernel,
        grid_spec=pltpu.PrefetchScalarGridSpec(
            num_scalar_prefetch=4,
            grid=(G,),
            in_specs=[
                pl.BlockSpec((BM, d), lambda g, se, sb, ss, sen: (sb[g], 0)),
                pl.BlockSpec((1, d, DFF),
                             lambda g, se, sb, ss, sen: (se[g], 0, 0)),
                pl.BlockSpec((1, DFF, d),
                             lambda g, se, sb, ss, sen: (se[g], 0, 0)),
            ],
            out_specs=pl.BlockSpec((BM, d),
                                   lambda g, se, sb, ss, sen: (sb[g], 0)),
        ),
        out_shape=jax.ShapeDtypeStruct((P, d), jnp.float32),
        compiler_params=pltpu.CompilerParams(
            dimension_semantics=("arbitrary",)),
    )(se, sb, ss, sen, x_sorted, W1, W2)

    rows = jnp.take(out_sorted, inv, axis=0).reshape(N, TOPK, d)
    y = jnp.sum(rows * wp.reshape(N, TOPK, 1), axis=1)
    return (y.reshape(b, t, d), aux[0, 0])


# speedup vs baseline: 1.2717x; 1.2717x over previous
import functools

import jax
import jax.numpy as jnp
from jax import lax
from jax.experimental import pallas as pl
from jax.experimental.pallas import tpu as pltpu
from jax.experimental.pallas import tpu_sc as plsc

TOPK = 2
BT = 256
BM = 128
NC = 2
NS = 16
NW = NC * NS


def _gate_kernel(h_ref, wg_ref, e1_ref, e2_ref, w1_ref, w2_ref,
                 p1_ref, p2_ref, loadw_ref, cnt_ref, aux_ref):
    g = pl.program_id(0)
    ng = pl.num_programs(0)
    E = wg_ref.shape[1]
    logits = jnp.dot(h_ref[...], wg_ref[...],
                     preferred_element_type=jnp.float32)
    eidx = jax.lax.broadcasted_iota(jnp.int32, logits.shape, 1)
    m1 = jnp.max(logits, axis=1, keepdims=True)
    a1 = jnp.min(jnp.where(logits == m1, eidx, E), axis=1, keepdims=True)
    masked = jnp.where(eidx == a1, -jnp.inf, logits)
    m2 = jnp.max(masked, axis=1, keepdims=True)
    a2 = jnp.min(jnp.where(masked == m2, eidx, E), axis=1, keepdims=True)
    p1 = 1.0 / (1.0 + jnp.exp(m2 - m1))
    p2 = 1.0 - p1
    e1_ref[...] = a1
    e2_ref[...] = a2
    w1_ref[...] = p1
    w2_ref[...] = p2
    one1 = (eidx == a1).astype(jnp.float32)
    one2 = (eidx == a2).astype(jnp.float32)
    both = one1 + one2
    ridx = jax.lax.broadcasted_iota(jnp.int32, (h_ref.shape[0],) * 2, 0)
    cidx = jax.lax.broadcasted_iota(jnp.int32, (h_ref.shape[0],) * 2, 1)
    tri = (cidx < ridx).astype(jnp.float32)
    csum = jnp.dot(tri, both, preferred_element_type=jnp.float32)
    carry = jnp.where(g == 0, 0.0, cnt_ref[...])
    total = carry + csum
    p1_ref[...] = jnp.sum(one1 * total, axis=1, keepdims=True).astype(
        jnp.int32)
    p2_ref[...] = jnp.sum(one2 * total, axis=1, keepdims=True).astype(
        jnp.int32)
    cnt_ref[...] = carry + jnp.sum(both, axis=0, keepdims=True)
    loadc = jnp.sum(one1 * p1 + one2 * p2, axis=0, keepdims=True)

    @pl.when(g == 0)
    def _():
        loadw_ref[...] = loadc

    @pl.when(g != 0)
    def _():
        loadw_ref[...] += loadc

    @pl.when(g == ng - 1)
    def _():
        load = loadw_ref[...]
        ln = load / jnp.sum(load)
        aux_ref[...] = jnp.sum(ln * jnp.log(ln + 1e-9)).reshape(1, 1)


def _ffn_kernel(se_ref, sb_ref, ss_ref, sen_ref, x_ref, w1_ref, w2_ref,
                o_ref):
    g = pl.program_id(0)
    b = sb_ref[g]
    start = ss_ref[g]
    end = sen_ref[g]
    r = b * BM + jax.lax.broadcasted_iota(jnp.int32, (BM, 1), 0)
    valid = (r >= start) & (r < end)
    xb = x_ref[...]
    h1 = jnp.dot(xb, w1_ref[0], preferred_element_type=jnp.float32)
    a = h1 * jax.nn.sigmoid(h1)
    z = jnp.dot(a, w2_ref[0], preferred_element_type=jnp.float32)
    z = jnp.where(valid, z, 0.0)
    first = jnp.logical_or(g == 0, sb_ref[jnp.maximum(g - 1, 0)] != b)

    @pl.when(first)
    def _():
        o_ref[...] = z

    @pl.when(jnp.logical_not(first))
    def _():
        o_ref[...] += z


def _make_dispatch(N, d, P):
    CT = N // NW
    mesh = plsc.VectorSubcoreMesh(core_axis_name="c", subcore_axis_name="s")

    @functools.partial(
        pl.kernel,
        out_type=jax.ShapeDtypeStruct((P, d), jnp.float32),
        mesh=mesh,
        scratch_types=[
            pltpu.VMEM((CT,), jnp.int32),
            pltpu.VMEM((CT,), jnp.int32),
            pltpu.VMEM((CT, d), jnp.float32),
            pltpu.SemaphoreType.DMA,
            pltpu.SemaphoreType.DMA,
        ],
    )
    def disp(h_hbm, inv0_hbm, inv1_hbm, xs_hbm, idx0_v, idx1_v, rows_v,
             sem0, sem1):
        w = lax.axis_index("s") * NC + lax.axis_index("c")
        base = w * CT
        pltpu.sync_copy(inv0_hbm.at[pl.ds(base, CT)], idx0_v)
        pltpu.sync_copy(inv1_hbm.at[pl.ds(base, CT)], idx1_v)
        pltpu.sync_copy(h_hbm.at[pl.ds(base, CT)], rows_v)
        c0 = pltpu.async_copy(rows_v, xs_hbm.at[idx0_v], sem0)
        c1 = pltpu.async_copy(rows_v, xs_hbm.at[idx1_v], sem1)
        c0.wait()
        c1.wait()

    return disp


def _make_combine(N, d, P):
    CT = N // NW
    NV = d // 16
    mesh = plsc.VectorSubcoreMesh(core_axis_name="c", subcore_axis_name="s")

    @functools.partial(
        pl.kernel,
        out_type=jax.ShapeDtypeStruct((N, d), jnp.float32),
        mesh=mesh,
        scratch_types=[
            pltpu.VMEM((CT,), jnp.int32),
            pltpu.VMEM((CT,), jnp.int32),
            pltpu.VMEM((CT, 16), jnp.float32),
            pltpu.VMEM((CT, 16), jnp.float32),
            pltpu.VMEM((CT, d), jnp.float32),
            pltpu.VMEM((CT, d), jnp.float32),
            pltpu.SemaphoreType.DMA,
            pltpu.SemaphoreType.DMA,
        ],
    )
    def comb(os_hbm, inv0_hbm, inv1_hbm, w0_hbm, w1_hbm, y_hbm,
             idx0_v, idx1_v, w0_v, w1_v, r0_v, r1_v, sem0, sem1):
        w = lax.axis_index("s") * NC + lax.axis_index("c")
        base = w * CT
        pltpu.sync_copy(inv0_hbm.at[pl.ds(base, CT)], idx0_v)
        pltpu.sync_copy(inv1_hbm.at[pl.ds(base, CT)], idx1_v)
        pltpu.sync_copy(w0_hbm.at[pl.ds(base, CT)], w0_v)
        pltpu.sync_copy(w1_hbm.at[pl.ds(base, CT)], w1_v)
        c0 = pltpu.async_copy(os_hbm.at[idx0_v], r0_v, sem0)
        c1 = pltpu.async_copy(os_hbm.at[idx1_v], r1_v, sem1)
        c0.wait()
        c1.wait()

        def body(t, carry):
            a = w0_v[t, :]
            c = w1_v[t, :]
            for j in range(NV):
                sl = pl.ds(j * 16, 16)
                r0_v[t, sl] = r0_v[t, sl] * a + r1_v[t, sl] * c
            return carry

        lax.fori_loop(0, CT, body, 0)
        pltpu.sync_copy(r0_v, y_hbm.at[pl.ds(base, CT)])

    return comb


def kernel(x, Wg, W1, W2):
    b, t, d = x.shape
    h = x.reshape(-1, d)
    N = h.shape[0]
    E = Wg.shape[1]
    DFF = W1.shape[2]
    P = N * TOPK
    NB = P // BM
    G = NB + E - 1

    ngate = N // BT
    e1, e2, w1, w2, pp1, pp2, loadw, cnt, aux = pl.pallas_call(
        _gate_kernel,
        grid=(ngate,),
        in_specs=[
            pl.BlockSpec((BT, d), lambda g: (g, 0)),
            pl.BlockSpec((d, E), lambda g: (0, 0)),
        ],
        out_specs=[
            pl.BlockSpec((BT, 1), lambda g: (g, 0)),
            pl.BlockSpec((BT, 1), lambda g: (g, 0)),
            pl.BlockSpec((BT, 1), lambda g: (g, 0)),
            pl.BlockSpec((BT, 1), lambda g: (g, 0)),
            pl.BlockSpec((BT, 1), lambda g: (g, 0)),
            pl.BlockSpec((BT, 1), lambda g: (g, 0)),
            pl.BlockSpec((1, E), lambda g: (0, 0)),
            pl.BlockSpec((1, E), lambda g: (0, 0)),
            pl.BlockSpec((1, 1), lambda g: (0, 0)),
        ],
        out_shape=[
            jax.ShapeDtypeStruct((N, 1), jnp.int32),
            jax.ShapeDtypeStruct((N, 1), jnp.int32),
            jax.ShapeDtypeStruct((N, 1), jnp.float32),
            jax.ShapeDtypeStruct((N, 1), jnp.float32),
            jax.ShapeDtypeStruct((N, 1), jnp.int32),
            jax.ShapeDtypeStruct((N, 1), jnp.int32),
            jax.ShapeDtypeStruct((1, E), jnp.float32),
            jax.ShapeDtypeStruct((1, E), jnp.float32),
            jax.ShapeDtypeStruct((1, 1), jnp.float32),
        ],
    )(h, Wg)

    counts = cnt[0].astype(jnp.int32)
    off = jnp.concatenate([jnp.zeros((1,), jnp.int32),
                           jnp.cumsum(counts)]).astype(jnp.int32)

    inv0 = (off[e1[:, 0]] + pp1[:, 0])
    inv1 = (off[e2[:, 0]] + pp2[:, 0])
    x_sorted = _make_dispatch(N, d, P)(h, inv0, inv1)

    first_blk = off[:-1] // BM
    last_blk = jnp.maximum(off[1:] - 1, 0) // BM
    nb = jnp.where(counts > 0, last_blk - first_blk + 1, 0)
    cum = jnp.cumsum(nb)
    total = cum[-1]
    gidx = jnp.arange(G, dtype=jnp.int32)
    eg = jnp.searchsorted(cum, gidx, side='right').astype(jnp.int32)
    eg = jnp.minimum(eg, E - 1)
    cum0 = jnp.concatenate([jnp.zeros((1,), jnp.int32),
                            cum.astype(jnp.int32)])
    j = gidx - cum0[eg]
    blk = first_blk[eg] + j
    live = gidx < total
    last_real = jnp.maximum(total - 1, 0)
    se = jnp.where(live, eg, eg[last_real]).astype(jnp.int32)
    sb = jnp.where(live, blk, blk[last_real]).astype(jnp.int32)
    ss = jnp.where(live, jnp.maximum(off[se], sb * BM), 0).astype(jnp.int32)
    sen = jnp.where(live, jnp.minimum(off[se + 1], (sb + 1) * BM),
                    0).astype(jnp.int32)

    out_sorted = pl.pallas_call(
        _ffn_kernel,
        grid_spec=pltpu.PrefetchScalarGridSpec(
            num_scalar_prefetch=4,
            grid=(G,),
            in_specs=[
                pl.BlockSpec((BM, d), lambda g, se, sb, ss, sen: (sb[g], 0)),
                pl.BlockSpec((1, d, DFF),
                             lambda g, se, sb, ss, sen: (se[g], 0, 0)),
                pl.BlockSpec((1, DFF, d),
                             lambda g, se, sb, ss, sen: (se[g], 0, 0)),
            ],
            out_specs=pl.BlockSpec((BM, d),
                                   lambda g, se, sb, ss, sen: (sb[g], 0)),
        ),
        out_shape=jax.ShapeDtypeStruct((P, d), jnp.float32),
        compiler_params=pltpu.CompilerParams(
            dimension_semantics=("arbitrary",)),
    )(se, sb, ss, sen, x_sorted, W1, W2)

    w0b = jnp.broadcast_to(w1, (N, 16))
    w1b = jnp.broadcast_to(w2, (N, 16))
    y = _make_combine(N, d, P)(out_sorted, inv0, inv1, w0b, w1b)
    return (y.reshape(b, t, d), aux[0, 0])


# speedup vs baseline: 1.4378x; 1.1306x over previous
import functools

import jax
import jax.numpy as jnp
from jax import lax
from jax.experimental import pallas as pl
from jax.experimental.pallas import tpu as pltpu
from jax.experimental.pallas import tpu_sc as plsc

TOPK = 2
BT = 256
BM = 128
NC = 2
NS = 16
NW = NC * NS


GPAD = 128


def _gate_kernel(h_ref, wg_ref,
                 inv0_ref, inv1_ref, w0b_ref, w1b_ref,
                 se_ref, sb_ref, ss_ref, sen_ref, aux_ref,
                 e1_s, e2_s, cnt_s, loadw_s):
    g = pl.program_id(0)
    ng = pl.num_programs(0)
    BTL = h_ref.shape[0]
    E = wg_ref.shape[1]
    hi = jax.lax.Precision.HIGHEST
    logits = jnp.dot(h_ref[...], wg_ref[...],
                     preferred_element_type=jnp.float32)
    eidx = jax.lax.broadcasted_iota(jnp.int32, logits.shape, 1)
    m1 = jnp.max(logits, axis=1, keepdims=True)
    a1 = jnp.min(jnp.where(logits == m1, eidx, E), axis=1, keepdims=True)
    masked = jnp.where(eidx == a1, -jnp.inf, logits)
    m2 = jnp.max(masked, axis=1, keepdims=True)
    a2 = jnp.min(jnp.where(masked == m2, eidx, E), axis=1, keepdims=True)
    p1 = 1.0 / (1.0 + jnp.exp(m2 - m1))
    p2 = 1.0 - p1
    rows = pl.ds(g * BTL, BTL)
    e1_s[rows, :] = a1
    e2_s[rows, :] = a2
    w0b_ref[...] = jnp.broadcast_to(p1, (BTL, 16))
    w1b_ref[...] = jnp.broadcast_to(p2, (BTL, 16))
    one1 = (eidx == a1).astype(jnp.float32)
    one2 = (eidx == a2).astype(jnp.float32)
    both = one1 + one2
    ridx = jax.lax.broadcasted_iota(jnp.int32, (BTL, BTL), 0)
    cidx = jax.lax.broadcasted_iota(jnp.int32, (BTL, BTL), 1)
    tri = (cidx < ridx).astype(jnp.float32)
    csum = jnp.dot(tri, both, preferred_element_type=jnp.float32)
    carry = jnp.where(g == 0, 0.0, cnt_s[...])
    total = carry + csum
    inv0_ref[rows, :] = jnp.sum(one1 * total, axis=1, keepdims=True).astype(
        jnp.int32)
    inv1_ref[rows, :] = jnp.sum(one2 * total, axis=1, keepdims=True).astype(
        jnp.int32)
    cnt_s[...] = carry + jnp.sum(both, axis=0, keepdims=True)
    loadc = jnp.sum(one1 * p1 + one2 * p2, axis=0, keepdims=True)

    @pl.when(g == 0)
    def _():
        loadw_s[...] = loadc

    @pl.when(g != 0)
    def _():
        loadw_s[...] += loadc

    @pl.when(g == ng - 1)
    def _():
        load = loadw_s[...]
        ln = load / jnp.sum(load)
        aux_ref[...] = jnp.sum(ln * jnp.log(ln + 1e-9)).reshape(1, 1)

        cnt_row = cnt_s[...]
        eyeE = (jax.lax.broadcasted_iota(jnp.int32, (E, E), 0) ==
                jax.lax.broadcasted_iota(jnp.int32, (E, E), 1)
                ).astype(jnp.float32)
        triE = (jax.lax.broadcasted_iota(jnp.int32, (E, E), 1) <=
                jax.lax.broadcasted_iota(jnp.int32, (E, E), 0)
                ).astype(jnp.float32)
        cnt_col = jax.lax.dot_general(
            eyeE, cnt_row, (((1,), (1,)), ((), ())),
            precision=hi, preferred_element_type=jnp.float32)
        off_incl = jnp.dot(triE, cnt_col, precision=hi,
                           preferred_element_type=jnp.float32)
        off_excl = off_incl - cnt_col
        first_blk = jnp.floor(off_excl * (1.0 / BM))
        last_blk = jnp.floor((off_incl - 1.0) * (1.0 / BM))
        nbv = jnp.where(cnt_col > 0, last_blk - first_blk + 1.0, 0.0)
        cum_col = jnp.dot(triE, nbv, precision=hi,
                          preferred_element_type=jnp.float32)
        cum_excl = cum_col - nbv
        tot = jnp.sum(nbv)
        gidx = jax.lax.broadcasted_iota(
            jnp.int32, (1, GPAD), 1).astype(jnp.float32)
        live = gidx < tot
        eg = jnp.sum((cum_col <= gidx).astype(jnp.float32), axis=0,
                     keepdims=True)
        eg = jnp.minimum(eg, float(E - 1))
        sel = (jax.lax.broadcasted_iota(jnp.int32, (E, GPAD), 0)
               .astype(jnp.float32) == eg
               ).astype(jnp.float32)
        cum_excl_at = jnp.sum(sel * cum_excl, axis=0, keepdims=True)
        first_at = jnp.sum(sel * first_blk, axis=0, keepdims=True)
        offE_at = jnp.sum(sel * off_excl, axis=0, keepdims=True)
        offI_at = jnp.sum(sel * off_incl, axis=0, keepdims=True)
        blk = first_at + (gidx - cum_excl_at)
        ssv = jnp.maximum(offE_at, blk * BM)
        senv = jnp.minimum(offI_at, (blk + 1.0) * BM)
        islast = (gidx == (tot - 1.0)).astype(jnp.float32)
        se_last = jnp.sum(islast * eg)
        sb_last = jnp.sum(islast * blk)
        se_ref[...] = jnp.where(live, eg, se_last).astype(jnp.int32)
        sb_ref[...] = jnp.where(live, blk, sb_last).astype(jnp.int32)
        ss_ref[...] = jnp.where(live, ssv, 0.0).astype(jnp.int32)
        sen_ref[...] = jnp.where(live, senv, 0.0).astype(jnp.int32)

        NE = (e1_s.shape[0], E)
        eN = jax.lax.broadcasted_iota(jnp.int32, NE, 1)
        oh1 = (e1_s[...] == eN).astype(jnp.float32)
        oh2 = (e2_s[...] == eN).astype(jnp.float32)
        add0 = jnp.dot(oh1, off_excl, precision=hi,
                       preferred_element_type=jnp.float32)
        add1 = jnp.dot(oh2, off_excl, precision=hi,
                       preferred_element_type=jnp.float32)
        inv0_ref[...] = inv0_ref[...] + add0.astype(jnp.int32)
        inv1_ref[...] = inv1_ref[...] + add1.astype(jnp.int32)


def _ffn_kernel(se_ref, sb_ref, ss_ref, sen_ref, x_ref, w1_ref, w2_ref,
                o_ref):
    g = pl.program_id(0)
    b = sb_ref[g]
    start = ss_ref[g]
    end = sen_ref[g]
    r = b * BM + jax.lax.broadcasted_iota(jnp.int32, (BM, 1), 0)
    valid = (r >= start) & (r < end)
    xb = x_ref[...]
    h1 = jnp.dot(xb, w1_ref[0], preferred_element_type=jnp.float32)
    a = h1 * jax.nn.sigmoid(h1)
    z = jnp.dot(a, w2_ref[0], preferred_element_type=jnp.float32)
    z = jnp.where(valid, z, 0.0)
    first = jnp.logical_or(g == 0, sb_ref[jnp.maximum(g - 1, 0)] != b)

    @pl.when(first)
    def _():
        o_ref[...] = z

    @pl.when(jnp.logical_not(first))
    def _():
        o_ref[...] += z


def _make_dispatch(N, d, P):
    CT = N // NW
    mesh = plsc.VectorSubcoreMesh(core_axis_name="c", subcore_axis_name="s")

    @functools.partial(
        pl.kernel,
        out_type=jax.ShapeDtypeStruct((P, d), jnp.float32),
        mesh=mesh,
        scratch_types=[
            pltpu.VMEM((CT,), jnp.int32),
            pltpu.VMEM((CT,), jnp.int32),
            pltpu.VMEM((CT, d), jnp.float32),
            pltpu.SemaphoreType.DMA,
            pltpu.SemaphoreType.DMA,
        ],
    )
    def disp(h_hbm, inv0_hbm, inv1_hbm, xs_hbm, idx0_v, idx1_v, rows_v,
             sem0, sem1):
        w = lax.axis_index("s") * NC + lax.axis_index("c")
        base = w * CT
        pltpu.sync_copy(inv0_hbm.at[pl.ds(base, CT)], idx0_v)
        pltpu.sync_copy(inv1_hbm.at[pl.ds(base, CT)], idx1_v)
        pltpu.sync_copy(h_hbm.at[pl.ds(base, CT)], rows_v)
        c0 = pltpu.async_copy(rows_v, xs_hbm.at[idx0_v], sem0)
        c1 = pltpu.async_copy(rows_v, xs_hbm.at[idx1_v], sem1)
        c0.wait()
        c1.wait()

    return disp


def _make_combine(N, d, P):
    CT = N // NW
    NV = d // 16
    mesh = plsc.VectorSubcoreMesh(core_axis_name="c", subcore_axis_name="s")

    @functools.partial(
        pl.kernel,
        out_type=jax.ShapeDtypeStruct((N, d), jnp.float32),
        mesh=mesh,
        scratch_types=[
            pltpu.VMEM((CT,), jnp.int32),
            pltpu.VMEM((CT,), jnp.int32),
            pltpu.VMEM((CT, 16), jnp.float32),
            pltpu.VMEM((CT, 16), jnp.float32),
            pltpu.VMEM((CT, d), jnp.float32),
            pltpu.VMEM((CT, d), jnp.float32),
            pltpu.SemaphoreType.DMA,
            pltpu.SemaphoreType.DMA,
        ],
    )
    def comb(os_hbm, inv0_hbm, inv1_hbm, w0_hbm, w1_hbm, y_hbm,
             idx0_v, idx1_v, w0_v, w1_v, r0_v, r1_v, sem0, sem1):
        w = lax.axis_index("s") * NC + lax.axis_index("c")
        base = w * CT
        pltpu.sync_copy(inv0_hbm.at[pl.ds(base, CT)], idx0_v)
        pltpu.sync_copy(inv1_hbm.at[pl.ds(base, CT)], idx1_v)
        pltpu.sync_copy(w0_hbm.at[pl.ds(base, CT)], w0_v)
        pltpu.sync_copy(w1_hbm.at[pl.ds(base, CT)], w1_v)
        c0 = pltpu.async_copy(os_hbm.at[idx0_v], r0_v, sem0)
        c1 = pltpu.async_copy(os_hbm.at[idx1_v], r1_v, sem1)
        c0.wait()
        c1.wait()

        def body(t, carry):
            a = w0_v[t, :]
            c = w1_v[t, :]
            for j in range(NV):
                sl = pl.ds(j * 16, 16)
                r0_v[t, sl] = r0_v[t, sl] * a + r1_v[t, sl] * c
            return carry

        lax.fori_loop(0, CT, body, 0)
        pltpu.sync_copy(r0_v, y_hbm.at[pl.ds(base, CT)])

    return comb


def kernel(x, Wg, W1, W2):
    b, t, d = x.shape
    h = x.reshape(-1, d)
    N = h.shape[0]
    E = Wg.shape[1]
    DFF = W1.shape[2]
    P = N * TOPK
    NB = P // BM
    G = NB + E - 1

    ngate = N // BT
    inv0c, inv1c, w0b, w1b, se_p, sb_p, ss_p, sen_p, aux = pl.pallas_call(
        _gate_kernel,
        grid=(ngate,),
        in_specs=[
            pl.BlockSpec((BT, d), lambda g: (g, 0)),
            pl.BlockSpec((d, E), lambda g: (0, 0)),
        ],
        out_specs=[
            pl.BlockSpec((N, 1), lambda g: (0, 0)),
            pl.BlockSpec((N, 1), lambda g: (0, 0)),
            pl.BlockSpec((BT, 16), lambda g: (g, 0)),
            pl.BlockSpec((BT, 16), lambda g: (g, 0)),
            pl.BlockSpec((1, GPAD), lambda g: (0, 0)),
            pl.BlockSpec((1, GPAD), lambda g: (0, 0)),
            pl.BlockSpec((1, GPAD), lambda g: (0, 0)),
            pl.BlockSpec((1, GPAD), lambda g: (0, 0)),
            pl.BlockSpec((1, 1), lambda g: (0, 0)),
        ],
        out_shape=[
            jax.ShapeDtypeStruct((N, 1), jnp.int32),
            jax.ShapeDtypeStruct((N, 1), jnp.int32),
            jax.ShapeDtypeStruct((N, 16), jnp.float32),
            jax.ShapeDtypeStruct((N, 16), jnp.float32),
            jax.ShapeDtypeStruct((1, GPAD), jnp.int32),
            jax.ShapeDtypeStruct((1, GPAD), jnp.int32),
            jax.ShapeDtypeStruct((1, GPAD), jnp.int32),
            jax.ShapeDtypeStruct((1, GPAD), jnp.int32),
            jax.ShapeDtypeStruct((1, 1), jnp.float32),
        ],
        scratch_shapes=[
            pltpu.VMEM((N, 1), jnp.int32),
            pltpu.VMEM((N, 1), jnp.int32),
            pltpu.VMEM((1, E), jnp.float32),
            pltpu.VMEM((1, E), jnp.float32),
        ],
    )(h, Wg)

    inv0 = inv0c[:, 0]
    inv1 = inv1c[:, 0]
    x_sorted = _make_dispatch(N, d, P)(h, inv0, inv1)

    se = se_p[0, :G]
    sb = sb_p[0, :G]
    ss = ss_p[0, :G]
    sen = sen_p[0, :G]

    out_sorted = pl.pallas_call(
        _ffn_kernel,
        grid_spec=pltpu.PrefetchScalarGridSpec(
            num_scalar_prefetch=4,
            grid=(G,),
            in_specs=[
                pl.BlockSpec((BM, d), lambda g, se, sb, ss, sen: (sb[g], 0)),
                pl.BlockSpec((1, d, DFF),
                             lambda g, se, sb, ss, sen: (se[g], 0, 0)),
                pl.BlockSpec((1, DFF, d),
                             lambda g, se, sb, ss, sen: (se[g], 0, 0)),
            ],
            out_specs=pl.BlockSpec((BM, d),
                                   lambda g, se, sb, ss, sen: (sb[g], 0)),
        ),
        out_shape=jax.ShapeDtypeStruct((P, d), jnp.float32),
        compiler_params=pltpu.CompilerParams(
            dimension_semantics=("arbitrary",)),
    )(se, sb, ss, sen, x_sorted, W1, W2)

    y = _make_combine(N, d, P)(out_sorted, inv0, inv1, w0b, w1b)
    return (y.reshape(b, t, d), aux[0, 0])
